# Initial kernel scaffold; baseline (speedup 1.0000x reference)
import functools

import jax
import jax.numpy as jnp
from jax import lax
from jax.experimental import pallas as pl
from jax.experimental.pallas import tpu as pltpu
from jax.experimental.pallas import tpu_sc as plsc

N = 10000
E = 160000
CHUNK = 128
NT = 16
NC = 2
EPAD = 163840
NCH = EPAD // (NT * CHUNK)
NCH_D = EPAD // (NT * NC * CHUNK)
NACC = N + 16
RPT = NACC // NT
DPT = N // NT
MB = 1000


def _make_deg_kernel():
    mesh = plsc.VectorSubcoreMesh(core_axis_name="c", subcore_axis_name="s")

    @functools.partial(
        pl.kernel,
        mesh=mesh,
        out_type=jax.ShapeDtypeStruct((NC * NACC, 16), jnp.float32),
        scratch_types=[
            pltpu.VMEM((NCH_D, CHUNK), jnp.int32),
            pltpu.VMEM((CHUNK, 16), jnp.float32),
            pltpu.VMEM_SHARED((NACC, 16), jnp.float32),
        ],
    )
    def deg_kernel(dst_hbm, ones_hbm, zeros_hbm, out_hbm, dst_v, ones_v, acc_sh):
        cid = lax.axis_index("c")
        sid = lax.axis_index("s")
        wid = sid * NC + cid
        pltpu.sync_copy(dst_hbm.at[pl.ds(wid * NCH_D, NCH_D)], dst_v)
        pltpu.sync_copy(ones_hbm, ones_v)
        pltpu.sync_copy(zeros_hbm.at[pl.ds(sid * RPT, RPT)],
                        acc_sh.at[pl.ds(sid * RPT, RPT)])
        plsc.subcore_barrier()

        def body(j, carry):
            pltpu.sync_copy(ones_v, acc_sh.at[dst_v.at[j]], add=True)
            return carry

        lax.fori_loop(0, NCH_D, body, 0)
        plsc.subcore_barrier()
        pltpu.sync_copy(acc_sh.at[pl.ds(sid * RPT, RPT)],
                        out_hbm.at[pl.ds(cid * NACC + sid * RPT, RPT)])

    return deg_kernel


def _make_prop_kernel(n_slices):
    kpc = n_slices // NC
    mesh = plsc.VectorSubcoreMesh(core_axis_name="c", subcore_axis_name="s")

    @functools.partial(
        pl.kernel,
        mesh=mesh,
        out_type=jax.ShapeDtypeStruct((n_slices * N, CHUNK), jnp.float32),
        scratch_types=[
            pltpu.VMEM((NCH, CHUNK), jnp.int32),
            pltpu.VMEM((NCH, CHUNK), jnp.int32),
            pltpu.VMEM((CHUNK, CHUNK), jnp.float32),
            pltpu.VMEM_SHARED((NACC, CHUNK), jnp.float32),
            pltpu.SemaphoreType.DMA,
        ],
    )
    def prop(y_hbm, srcg_hbm, dst_hbm, zeros_hbm, out_hbm,
             src_v, dst_v, gbuf, acc_sh, sem):
        cid = lax.axis_index("c")
        sid = lax.axis_index("s")
        pltpu.sync_copy(dst_hbm.at[pl.ds(sid * NCH, NCH)], dst_v)
        for k in range(kpc):
            si = k * NC + cid
            pltpu.sync_copy(
                srcg_hbm.at[pl.ds(si * (NT * NCH) + sid * NCH, NCH)], src_v)
            pltpu.sync_copy(zeros_hbm.at[pl.ds(sid * RPT, RPT)],
                            acc_sh.at[pl.ds(sid * RPT, RPT)])
            plsc.subcore_barrier()

            def body(j, carry):
                pltpu.async_copy(y_hbm.at[src_v.at[j]], gbuf, sem).wait()
                pltpu.sync_copy(gbuf, acc_sh.at[dst_v.at[j]], add=True)
                return carry

            lax.fori_loop(0, NCH, body, 0)
            plsc.subcore_barrier()
            pltpu.sync_copy(acc_sh.at[pl.ds(sid * DPT, DPT)],
                            out_hbm.at[pl.ds(si * N + sid * DPT, DPT)])
            plsc.subcore_barrier()

    return prop


def _scale(x, d0, d1):
    def body(x_ref, d0_ref, d1_ref, o_ref):
        deg = d0_ref[...][:, 0:1] + d1_ref[...][:, 0:1] + 1.0
        o_ref[...] = x_ref[...] * lax.rsqrt(deg)

    return pl.pallas_call(
        body,
        grid=(2, N // MB),
        in_specs=[
            pl.BlockSpec((MB, CHUNK), lambda si, i: (i, si)),
            pl.BlockSpec((MB, 16), lambda si, i: (i, 0)),
            pl.BlockSpec((MB, 16), lambda si, i: (i, 0)),
        ],
        out_specs=pl.BlockSpec((MB, CHUNK), lambda si, i: (si * (N // MB) + i, 0)),
        out_shape=jax.ShapeDtypeStruct((2 * N, CHUNK), jnp.float32),
    )(x, d0, d1)


def _layer1(S1, y1, d0, d1, W1, b1):
    def body(s_ref, y_ref, d0_ref, d1_ref, w_ref, b_ref, o_ref):
        deg = d0_ref[...][:, 0:1] + d1_ref[...][:, 0:1] + 1.0
        dinv = lax.rsqrt(deg)
        p = jnp.concatenate(
            [s_ref[0] + y_ref[0], s_ref[1] + y_ref[1]], axis=1) * dinv
        h = jnp.dot(p, w_ref[...], preferred_element_type=jnp.float32)
        h = jnp.maximum(h + b_ref[...], 0.0) * dinv
        for k in range(4):
            o_ref[k] = h[:, k * CHUNK:(k + 1) * CHUNK]

    return pl.pallas_call(
        body,
        grid=(N // MB,),
        in_specs=[
            pl.BlockSpec((2, MB, CHUNK), lambda i: (0, i, 0)),
            pl.BlockSpec((2, MB, CHUNK), lambda i: (0, i, 0)),
            pl.BlockSpec((MB, 16), lambda i: (i, 0)),
            pl.BlockSpec((MB, 16), lambda i: (i, 0)),
            pl.BlockSpec((256, 512), lambda i: (0, 0)),
            pl.BlockSpec((1, 512), lambda i: (0, 0)),
        ],
        out_specs=pl.BlockSpec((4, MB, CHUNK), lambda i: (0, i, 0)),
        out_shape=jax.ShapeDtypeStruct((4, N, CHUNK), jnp.float32),
    )(S1, y1, d0, d1, W1, b1)


def _final(S2, y2, d0, d1, Wc, bc):
    def body(s_ref, y_ref, d0_ref, d1_ref, w_ref, b_ref, o_ref):
        deg = d0_ref[...][:, 0:1] + d1_ref[...][:, 0:1] + 1.0
        dinv = lax.rsqrt(deg)
        p = jnp.concatenate(
            [s_ref[k] + y_ref[k] for k in range(4)], axis=1) * dinv
        o_ref[...] = jnp.dot(
            p, w_ref[...], preferred_element_type=jnp.float32) + b_ref[...]

    return pl.pallas_call(
        body,
        grid=(N // MB,),
        in_specs=[
            pl.BlockSpec((4, MB, CHUNK), lambda i: (0, i, 0)),
            pl.BlockSpec((4, MB, CHUNK), lambda i: (0, i, 0)),
            pl.BlockSpec((MB, 16), lambda i: (i, 0)),
            pl.BlockSpec((MB, 16), lambda i: (i, 0)),
            pl.BlockSpec((512, 512), lambda i: (0, 0)),
            pl.BlockSpec((1, 512), lambda i: (0, 0)),
        ],
        out_specs=pl.BlockSpec((MB, 512), lambda i: (i, 0)),
        out_shape=jax.ShapeDtypeStruct((N, 512), jnp.float32),
    )(S2, y2, d0, d1, Wc, bc)


def kernel(x, edge_index, W1, b1, W_mu, b_mu, W_ls, b_ls):
    src = edge_index[0].astype(jnp.int32)
    dst = edge_index[1].astype(jnp.int32)
    pad = EPAD - E
    padi = jnp.arange(pad, dtype=jnp.int32) % 16
    src_p = jnp.concatenate([src, padi])
    dst_p = jnp.concatenate([dst, N + padi])
    dst_flat = dst_p.reshape(EPAD // CHUNK, CHUNK)
    ones16 = jnp.ones((CHUNK, 16), jnp.float32)
    zeros16 = jnp.zeros((NACC, 16), jnp.float32)
    zeros128 = jnp.zeros((NACC, CHUNK), jnp.float32)

    degP = _make_deg_kernel()(dst_flat, ones16, zeros16)
    d0 = degP[:N]
    d1 = degP[NACC:NACC + N]

    y1 = _scale(x, d0, d1)

    off2 = (jnp.arange(2, dtype=jnp.int32) * N)[:, None]
    src2 = (src_p[None, :] + off2).reshape(2 * EPAD // CHUNK, CHUNK)
    S1 = _make_prop_kernel(2)(y1, src2, dst_flat, zeros128)

    y2 = _layer1(S1.reshape(2, N, CHUNK), y1.reshape(2, N, CHUNK),
                 d0, d1, W1, b1.reshape(1, 512))

    off4 = (jnp.arange(4, dtype=jnp.int32) * N)[:, None]
    src4 = (src_p[None, :] + off4).reshape(4 * EPAD // CHUNK, CHUNK)
    S2 = _make_prop_kernel(4)(y2.reshape(4 * N, CHUNK), src4, dst_flat, zeros128)

    Wc = jnp.concatenate([W_mu, W_ls], axis=1)
    bc = jnp.concatenate([b_mu, b_ls]).reshape(1, 512)
    out = _final(S2.reshape(4, N, CHUNK), y2, d0, d1, Wc, bc)
    return out[:, :256], out[:, 256:]


# speedup vs baseline: 13.1032x; 13.1032x over previous
import functools

import jax
import jax.numpy as jnp
from jax import lax
from jax.experimental import pallas as pl
from jax.experimental.pallas import tpu as pltpu
from jax.experimental.pallas import tpu_sc as plsc

N = 10000
E = 160000
CHUNK = 128
NT = 16
NC = 2
EPAD = 163840
NCH = EPAD // (NT * CHUNK)
NCH_D = EPAD // (NT * NC * CHUNK)
NACC = 10112
RPT = NACC // NT
DR = 632
DR_LAST = N - (NT - 1) * DR
MB = 1000


def _make_deg_kernel():
    mesh = plsc.VectorSubcoreMesh(core_axis_name="c", subcore_axis_name="s")

    @functools.partial(
        pl.kernel,
        mesh=mesh,
        out_type=jax.ShapeDtypeStruct((NC * NACC, CHUNK), jnp.float32),
        scratch_types=[
            pltpu.VMEM((NCH_D, CHUNK), jnp.int32),
            pltpu.VMEM((CHUNK, CHUNK), jnp.float32),
            pltpu.VMEM_SHARED((NACC, CHUNK), jnp.float32),
        ],
    )
    def deg_kernel(dst_hbm, ones_hbm, zeros_hbm, out_hbm, dst_v, ones_v, acc_sh):
        cid = lax.axis_index("c")
        sid = lax.axis_index("s")
        wid = sid * NC + cid
        pltpu.sync_copy(dst_hbm.at[pl.ds(wid * NCH_D, NCH_D)], dst_v)
        pltpu.sync_copy(ones_hbm, ones_v)
        pltpu.sync_copy(zeros_hbm.at[pl.ds(sid * RPT, RPT)],
                        acc_sh.at[pl.ds(sid * RPT, RPT)])
        plsc.subcore_barrier()

        def body(j, carry):
            pltpu.sync_copy(ones_v, acc_sh.at[dst_v.at[j]], add=True)
            return carry

        lax.fori_loop(0, NCH_D, body, 0)
        plsc.subcore_barrier()
        pltpu.sync_copy(acc_sh.at[pl.ds(sid * RPT, RPT)],
                        out_hbm.at[pl.ds(cid * NACC + sid * RPT, RPT)])

    return deg_kernel


def _make_prop_kernel(n_slices):
    kpc = n_slices // NC
    mesh = plsc.VectorSubcoreMesh(core_axis_name="c", subcore_axis_name="s")

    @functools.partial(
        pl.kernel,
        mesh=mesh,
        out_type=jax.ShapeDtypeStruct((n_slices * N, CHUNK), jnp.float32),
        scratch_types=[
            pltpu.VMEM((NCH, CHUNK), jnp.int32),
            pltpu.VMEM((NCH, CHUNK), jnp.int32),
            pltpu.VMEM((CHUNK, CHUNK), jnp.float32),
            pltpu.VMEM_SHARED((NACC, CHUNK), jnp.float32),
            pltpu.SemaphoreType.DMA,
        ],
    )
    def prop(y_hbm, srcg_hbm, dst_hbm, zeros_hbm, out_hbm,
             src_v, dst_v, gbuf, acc_sh, sem):
        cid = lax.axis_index("c")
        sid = lax.axis_index("s")
        pltpu.sync_copy(dst_hbm.at[pl.ds(sid * NCH, NCH)], dst_v)
        for k in range(kpc):
            si = k * NC + cid
            pltpu.sync_copy(
                srcg_hbm.at[pl.ds(si * (NT * NCH) + sid * NCH, NCH)], src_v)
            pltpu.sync_copy(zeros_hbm.at[pl.ds(sid * RPT, RPT)],
                            acc_sh.at[pl.ds(sid * RPT, RPT)])
            plsc.subcore_barrier()

            def body(j, carry):
                pltpu.async_copy(y_hbm.at[src_v.at[j]], gbuf, sem).wait()
                pltpu.sync_copy(gbuf, acc_sh.at[dst_v.at[j]], add=True)
                return carry

            lax.fori_loop(0, NCH, body, 0)
            plsc.subcore_barrier()

            @pl.when(sid < NT - 1)
            def _():
                pltpu.sync_copy(acc_sh.at[pl.ds(sid * DR, DR)],
                                out_hbm.at[pl.ds(si * N + sid * DR, DR)])

            @pl.when(sid == NT - 1)
            def _():
                pltpu.sync_copy(acc_sh.at[pl.ds((NT - 1) * DR, DR_LAST)],
                                out_hbm.at[pl.ds(si * N + (NT - 1) * DR, DR_LAST)])

            plsc.subcore_barrier()

    return prop


def _scale(x, d0, d1):
    def body(x_ref, d0_ref, d1_ref, o_ref):
        deg = d0_ref[...][:, 0:1] + d1_ref[...][:, 0:1] + 1.0
        o_ref[...] = x_ref[...] * lax.rsqrt(deg)

    return pl.pallas_call(
        body,
        grid=(2, N // MB),
        in_specs=[
            pl.BlockSpec((MB, CHUNK), lambda si, i: (i, si)),
            pl.BlockSpec((MB, 16), lambda si, i: (i, 0)),
            pl.BlockSpec((MB, 16), lambda si, i: (i, 0)),
        ],
        out_specs=pl.BlockSpec((MB, CHUNK), lambda si, i: (si * (N // MB) + i, 0)),
        out_shape=jax.ShapeDtypeStruct((2 * N, CHUNK), jnp.float32),
    )(x, d0, d1)


def _layer1(S1, y1, d0, d1, W1, b1):
    def body(s_ref, y_ref, d0_ref, d1_ref, w_ref, b_ref, o_ref):
        deg = d0_ref[...][:, 0:1] + d1_ref[...][:, 0:1] + 1.0
        dinv = lax.rsqrt(deg)
        p = jnp.concatenate(
            [s_ref[0] + y_ref[0], s_ref[1] + y_ref[1]], axis=1) * dinv
        h = jnp.dot(p, w_ref[...], preferred_element_type=jnp.float32)
        h = jnp.maximum(h + b_ref[...], 0.0) * dinv
        for k in range(4):
            o_ref[k] = h[:, k * CHUNK:(k + 1) * CHUNK]

    return pl.pallas_call(
        body,
        grid=(N // MB,),
        in_specs=[
            pl.BlockSpec((2, MB, CHUNK), lambda i: (0, i, 0)),
            pl.BlockSpec((2, MB, CHUNK), lambda i: (0, i, 0)),
            pl.BlockSpec((MB, 16), lambda i: (i, 0)),
            pl.BlockSpec((MB, 16), lambda i: (i, 0)),
            pl.BlockSpec((256, 512), lambda i: (0, 0)),
            pl.BlockSpec((1, 512), lambda i: (0, 0)),
        ],
        out_specs=pl.BlockSpec((4, MB, CHUNK), lambda i: (0, i, 0)),
        out_shape=jax.ShapeDtypeStruct((4, N, CHUNK), jnp.float32),
    )(S1, y1, d0, d1, W1, b1)


def _final(S2, y2, d0, d1, Wc, bc):
    def body(s_ref, y_ref, d0_ref, d1_ref, w_ref, b_ref, o_ref):
        deg = d0_ref[...][:, 0:1] + d1_ref[...][:, 0:1] + 1.0
        dinv = lax.rsqrt(deg)
        p = jnp.concatenate(
            [s_ref[k] + y_ref[k] for k in range(4)], axis=1) * dinv
        o_ref[...] = jnp.dot(
            p, w_ref[...], preferred_element_type=jnp.float32) + b_ref[...]

    return pl.pallas_call(
        body,
        grid=(N // MB,),
        in_specs=[
            pl.BlockSpec((4, MB, CHUNK), lambda i: (0, i, 0)),
            pl.BlockSpec((4, MB, CHUNK), lambda i: (0, i, 0)),
            pl.BlockSpec((MB, 16), lambda i: (i, 0)),
            pl.BlockSpec((MB, 16), lambda i: (i, 0)),
            pl.BlockSpec((512, 512), lambda i: (0, 0)),
            pl.BlockSpec((1, 512), lambda i: (0, 0)),
        ],
        out_specs=pl.BlockSpec((MB, 512), lambda i: (i, 0)),
        out_shape=jax.ShapeDtypeStruct((N, 512), jnp.float32),
    )(S2, y2, d0, d1, Wc, bc)


def kernel(x, edge_index, W1, b1, W_mu, b_mu, W_ls, b_ls):
    src = edge_index[0].astype(jnp.int32)
    dst = edge_index[1].astype(jnp.int32)
    pad = EPAD - E
    padi = jnp.arange(pad, dtype=jnp.int32) % 16
    src_p = jnp.concatenate([src, padi])
    dst_p = jnp.concatenate([dst, N + padi])
    dst_flat = dst_p.reshape(EPAD // CHUNK, CHUNK)
    ones128 = jnp.ones((CHUNK, CHUNK), jnp.float32)
    zeros128 = jnp.zeros((NACC, CHUNK), jnp.float32)

    degP = _make_deg_kernel()(dst_flat, ones128, zeros128)
    d0 = degP[:N, :16]
    d1 = degP[NACC:NACC + N, :16]

    y1 = _scale(x, d0, d1)

    off2 = (jnp.arange(2, dtype=jnp.int32) * N)[:, None]
    src2 = (src_p[None, :] + off2).reshape(2 * EPAD // CHUNK, CHUNK)
    S1 = _make_prop_kernel(2)(y1, src2, dst_flat, zeros128)

    y2 = _layer1(S1.reshape(2, N, CHUNK), y1.reshape(2, N, CHUNK),
                 d0, d1, W1, b1.reshape(1, 512))

    off4 = (jnp.arange(4, dtype=jnp.int32) * N)[:, None]
    src4 = (src_p[None, :] + off4).reshape(4 * EPAD // CHUNK, CHUNK)
    S2 = _make_prop_kernel(4)(y2.reshape(4 * N, CHUNK), src4, dst_flat, zeros128)

    Wc = jnp.concatenate([W_mu, W_ls], axis=1)
    bc = jnp.concatenate([b_mu, b_ls]).reshape(1, 512)
    out = _final(S2.reshape(4, N, CHUNK), y2, d0, d1, Wc, bc)
    return out[:, :256], out[:, 256:]


# speedup vs baseline: 18.1512x; 1.3853x over previous
import functools

import jax
import jax.numpy as jnp
from jax import lax
from jax.experimental import pallas as pl
from jax.experimental.pallas import tpu as pltpu
from jax.experimental.pallas import tpu_sc as plsc

N = 10000
E = 160000
CHUNK = 128
NT = 16
NC = 2
EPAD = 163840
NCH = EPAD // (NT * CHUNK)
NCH_D = EPAD // (NT * NC * CHUNK)
NACC = 10112
RPT = NACC // NT
DR = 632
DR_LAST = N - (NT - 1) * DR
MB = 1000


def _make_deg_kernel():
    mesh = plsc.VectorSubcoreMesh(core_axis_name="c", subcore_axis_name="s")

    @functools.partial(
        pl.kernel,
        mesh=mesh,
        out_type=jax.ShapeDtypeStruct((NC * NACC, CHUNK), jnp.float32),
        scratch_types=[
            pltpu.VMEM((NCH_D, CHUNK), jnp.int32),
            pltpu.VMEM((CHUNK, CHUNK), jnp.float32),
            pltpu.VMEM_SHARED((NACC, CHUNK), jnp.float32),
        ],
    )
    def deg_kernel(dst_hbm, ones_hbm, zeros_hbm, out_hbm, dst_v, ones_v, acc_sh):
        cid = lax.axis_index("c")
        sid = lax.axis_index("s")
        wid = sid * NC + cid
        pltpu.sync_copy(dst_hbm.at[pl.ds(wid * NCH_D, NCH_D)], dst_v)
        pltpu.sync_copy(ones_hbm, ones_v)
        pltpu.sync_copy(zeros_hbm.at[pl.ds(sid * RPT, RPT)],
                        acc_sh.at[pl.ds(sid * RPT, RPT)])
        plsc.subcore_barrier()

        def body(j, carry):
            pltpu.sync_copy(ones_v, acc_sh.at[dst_v.at[j]], add=True)
            return carry

        lax.fori_loop(0, NCH_D, body, 0)
        plsc.subcore_barrier()
        pltpu.sync_copy(acc_sh.at[pl.ds(sid * RPT, RPT)],
                        out_hbm.at[pl.ds(cid * NACC + sid * RPT, RPT)])

    return deg_kernel


def _make_prop_kernel(n_slices):
    kpc = n_slices // NC
    half = NCH // 2
    mesh = plsc.VectorSubcoreMesh(core_axis_name="c", subcore_axis_name="s")

    @functools.partial(
        pl.kernel,
        mesh=mesh,
        out_type=jax.ShapeDtypeStruct((n_slices * N, CHUNK), jnp.float32),
        scratch_types=[
            pltpu.VMEM((half, CHUNK), jnp.int32),
            pltpu.VMEM((half, CHUNK), jnp.int32),
            pltpu.VMEM((CHUNK, CHUNK), jnp.float32),
            pltpu.VMEM((CHUNK, CHUNK), jnp.float32),
            pltpu.VMEM_SHARED((NACC, CHUNK), jnp.float32),
            pltpu.SemaphoreType.DMA,
            pltpu.SemaphoreType.DMA,
        ],
    )
    def prop(y_hbm, srcg_hbm, dst_hbm, zeros_hbm, out_hbm,
             src_v, dst_v, gbuf0, gbuf1, acc_sh, sem0, sem1):
        cid = lax.axis_index("c")
        sid = lax.axis_index("s")
        for k in range(kpc):
            si = k * NC + cid
            pltpu.sync_copy(zeros_hbm.at[pl.ds(sid * RPT, RPT)],
                            acc_sh.at[pl.ds(sid * RPT, RPT)])
            plsc.subcore_barrier()

            for h in range(2):
                pltpu.sync_copy(
                    srcg_hbm.at[pl.ds(
                        si * (NT * NCH) + sid * NCH + h * half, half)],
                    src_v)
                pltpu.sync_copy(
                    dst_hbm.at[pl.ds(sid * NCH + h * half, half)], dst_v)

                pltpu.async_copy(y_hbm.at[src_v.at[0]], gbuf0, sem0)

                def body(i, carry):
                    j0 = 2 * i
                    pltpu.async_copy(y_hbm.at[src_v.at[j0 + 1]], gbuf1, sem1)
                    pltpu.make_async_copy(
                        y_hbm.at[src_v.at[j0]], gbuf0, sem0).wait()
                    pltpu.sync_copy(gbuf0, acc_sh.at[dst_v.at[j0]], add=True)

                    j2 = jnp.minimum(j0 + 2, half - 2)
                    pltpu.async_copy(y_hbm.at[src_v.at[j2]], gbuf0, sem0)

                    pltpu.make_async_copy(
                        y_hbm.at[src_v.at[j0 + 1]], gbuf1, sem1).wait()
                    pltpu.sync_copy(gbuf1, acc_sh.at[dst_v.at[j0 + 1]], add=True)
                    return carry

                lax.fori_loop(0, half // 2, body, 0)
                pltpu.make_async_copy(
                    y_hbm.at[src_v.at[half - 2]], gbuf0, sem0).wait()

            plsc.subcore_barrier()

            @pl.when(sid < NT - 1)
            def _():
                pltpu.sync_copy(acc_sh.at[pl.ds(sid * DR, DR)],
                                out_hbm.at[pl.ds(si * N + sid * DR, DR)])

            @pl.when(sid == NT - 1)
            def _():
                pltpu.sync_copy(acc_sh.at[pl.ds((NT - 1) * DR, DR_LAST)],
                                out_hbm.at[pl.ds(si * N + (NT - 1) * DR, DR_LAST)])

            plsc.subcore_barrier()

    return prop


def _scale(x, d0, d1):
    def body(x_ref, d0_ref, d1_ref, o_ref):
        deg = d0_ref[...][:, 0:1] + d1_ref[...][:, 0:1] + 1.0
        o_ref[...] = x_ref[...] * lax.rsqrt(deg)

    return pl.pallas_call(
        body,
        grid=(2, N // MB),
        in_specs=[
            pl.BlockSpec((MB, CHUNK), lambda si, i: (i, si)),
            pl.BlockSpec((MB, 16), lambda si, i: (i, 0)),
            pl.BlockSpec((MB, 16), lambda si, i: (i, 0)),
        ],
        out_specs=pl.BlockSpec((MB, CHUNK), lambda si, i: (si * (N // MB) + i, 0)),
        out_shape=jax.ShapeDtypeStruct((2 * N, CHUNK), jnp.float32),
    )(x, d0, d1)


def _layer1(S1, y1, d0, d1, W1, b1):
    def body(s_ref, y_ref, d0_ref, d1_ref, w_ref, b_ref, o_ref):
        deg = d0_ref[...][:, 0:1] + d1_ref[...][:, 0:1] + 1.0
        dinv = lax.rsqrt(deg)
        p = jnp.concatenate(
            [s_ref[0] + y_ref[0], s_ref[1] + y_ref[1]], axis=1) * dinv
        h = jnp.dot(p, w_ref[...], preferred_element_type=jnp.float32)
        h = jnp.maximum(h + b_ref[...], 0.0) * dinv
        for k in range(4):
            o_ref[k] = h[:, k * CHUNK:(k + 1) * CHUNK]

    return pl.pallas_call(
        body,
        grid=(N // MB,),
        in_specs=[
            pl.BlockSpec((2, MB, CHUNK), lambda i: (0, i, 0)),
            pl.BlockSpec((2, MB, CHUNK), lambda i: (0, i, 0)),
            pl.BlockSpec((MB, 16), lambda i: (i, 0)),
            pl.BlockSpec((MB, 16), lambda i: (i, 0)),
            pl.BlockSpec((256, 512), lambda i: (0, 0)),
            pl.BlockSpec((1, 512), lambda i: (0, 0)),
        ],
        out_specs=pl.BlockSpec((4, MB, CHUNK), lambda i: (0, i, 0)),
        out_shape=jax.ShapeDtypeStruct((4, N, CHUNK), jnp.float32),
    )(S1, y1, d0, d1, W1, b1)


def _final(S2, y2, d0, d1, Wc, bc):
    def body(s_ref, y_ref, d0_ref, d1_ref, w_ref, b_ref, o_ref):
        deg = d0_ref[...][:, 0:1] + d1_ref[...][:, 0:1] + 1.0
        dinv = lax.rsqrt(deg)
        p = jnp.concatenate(
            [s_ref[k] + y_ref[k] for k in range(4)], axis=1) * dinv
        o_ref[...] = jnp.dot(
            p, w_ref[...], preferred_element_type=jnp.float32) + b_ref[...]

    return pl.pallas_call(
        body,
        grid=(N // MB,),
        in_specs=[
            pl.BlockSpec((4, MB, CHUNK), lambda i: (0, i, 0)),
            pl.BlockSpec((4, MB, CHUNK), lambda i: (0, i, 0)),
            pl.BlockSpec((MB, 16), lambda i: (i, 0)),
            pl.BlockSpec((MB, 16), lambda i: (i, 0)),
            pl.BlockSpec((512, 512), lambda i: (0, 0)),
            pl.BlockSpec((1, 512), lambda i: (0, 0)),
        ],
        out_specs=pl.BlockSpec((MB, 512), lambda i: (i, 0)),
        out_shape=jax.ShapeDtypeStruct((N, 512), jnp.float32),
    )(S2, y2, d0, d1, Wc, bc)


def kernel(x, edge_index, W1, b1, W_mu, b_mu, W_ls, b_ls):
    src = edge_index[0].astype(jnp.int32)
    dst = edge_index[1].astype(jnp.int32)
    pad = EPAD - E
    padi = jnp.arange(pad, dtype=jnp.int32) % 16
    src_p = jnp.concatenate([src, padi])
    dst_p = jnp.concatenate([dst, N + padi])
    dst_flat = dst_p.reshape(EPAD // CHUNK, CHUNK)
    ones128 = jnp.ones((CHUNK, CHUNK), jnp.float32)
    zeros128 = jnp.zeros((NACC, CHUNK), jnp.float32)

    degP = _make_deg_kernel()(dst_flat, ones128, zeros128)
    d0 = degP[:N, :16]
    d1 = degP[NACC:NACC + N, :16]

    y1 = _scale(x, d0, d1)

    off2 = (jnp.arange(2, dtype=jnp.int32) * N)[:, None]
    src2 = (src_p[None, :] + off2).reshape(2 * EPAD // CHUNK, CHUNK)
    S1 = _make_prop_kernel(2)(y1, src2, dst_flat, zeros128)

    y2 = _layer1(S1.reshape(2, N, CHUNK), y1.reshape(2, N, CHUNK),
                 d0, d1, W1, b1.reshape(1, 512))

    off4 = (jnp.arange(4, dtype=jnp.int32) * N)[:, None]
    src4 = (src_p[None, :] + off4).reshape(4 * EPAD // CHUNK, CHUNK)
    S2 = _make_prop_kernel(4)(y2.reshape(4 * N, CHUNK), src4, dst_flat, zeros128)

    Wc = jnp.concatenate([W_mu, W_ls], axis=1)
    bc = jnp.concatenate([b_mu, b_ls]).reshape(1, 512)
    out = _final(S2.reshape(4, N, CHUNK), y2, d0, d1, Wc, bc)
    return out[:, :256], out[:, 256:]


# speedup vs baseline: 18.9682x; 1.0450x over previous
import functools

import jax
import jax.numpy as jnp
from jax import lax
from jax.experimental import pallas as pl
from jax.experimental.pallas import tpu as pltpu
from jax.experimental.pallas import tpu_sc as plsc

N = 10000
E = 160000
CHUNK = 128
NT = 16
NC = 2
EPAD = 163840
NCH = EPAD // (NT * CHUNK)
NCH_D = EPAD // (NT * NC * CHUNK)
NACC = 10112
RPT = NACC // NT
DR = 632
DR_LAST = N - (NT - 1) * DR
MB = 1000


def _make_deg_kernel():
    mesh = plsc.VectorSubcoreMesh(core_axis_name="c", subcore_axis_name="s")

    @functools.partial(
        pl.kernel,
        mesh=mesh,
        out_type=jax.ShapeDtypeStruct((NC * NACC, CHUNK), jnp.float32),
        scratch_types=[
            pltpu.VMEM((NCH_D, CHUNK), jnp.int32),
            pltpu.VMEM((CHUNK, CHUNK), jnp.float32),
            pltpu.VMEM_SHARED((NACC, CHUNK), jnp.float32),
        ],
    )
    def deg_kernel(dst_hbm, ones_hbm, zeros_hbm, out_hbm, dst_v, ones_v, acc_sh):
        cid = lax.axis_index("c")
        sid = lax.axis_index("s")
        wid = sid * NC + cid
        pltpu.sync_copy(dst_hbm.at[pl.ds(wid * NCH_D, NCH_D)], dst_v)
        pltpu.sync_copy(ones_hbm, ones_v)
        pltpu.sync_copy(zeros_hbm.at[pl.ds(sid * RPT, RPT)],
                        acc_sh.at[pl.ds(sid * RPT, RPT)])
        plsc.subcore_barrier()

        def body(j, carry):
            pltpu.sync_copy(ones_v, acc_sh.at[dst_v.at[j]], add=True)
            return carry

        lax.fori_loop(0, NCH_D, body, 0)
        plsc.subcore_barrier()
        pltpu.sync_copy(acc_sh.at[pl.ds(sid * RPT, RPT)],
                        out_hbm.at[pl.ds(cid * NACC + sid * RPT, RPT)])

    return deg_kernel


def _make_prop_kernel(n_slices):
    half = NCH // 2
    mesh = plsc.VectorSubcoreMesh(core_axis_name="c", subcore_axis_name="s")

    @functools.partial(
        pl.kernel,
        mesh=mesh,
        out_type=[jax.ShapeDtypeStruct((N, CHUNK), jnp.float32)] * n_slices,
        scratch_types=[
            pltpu.VMEM((half, CHUNK), jnp.int32),
            pltpu.VMEM((half, CHUNK), jnp.int32),
            pltpu.VMEM((CHUNK, CHUNK), jnp.float32),
            pltpu.VMEM((CHUNK, CHUNK), jnp.float32),
            pltpu.VMEM_SHARED((NACC, CHUNK), jnp.float32),
            pltpu.SemaphoreType.DMA,
            pltpu.SemaphoreType.DMA,
        ],
    )
    def prop(*refs):
        y_refs = refs[:n_slices]
        src_hbm, dst_hbm, zeros_hbm = refs[n_slices:n_slices + 3]
        out_refs = refs[n_slices + 3:2 * n_slices + 3]
        src_v, dst_v, gbuf0, gbuf1, acc_sh, sem0, sem1 = refs[2 * n_slices + 3:]
        cid = lax.axis_index("c")
        sid = lax.axis_index("s")
        for si in range(n_slices):

            @pl.when(cid == si % 2)
            def _(si=si):
                y_hbm = y_refs[si]
                out_hbm = out_refs[si]
                pltpu.sync_copy(zeros_hbm.at[pl.ds(sid * RPT, RPT)],
                                acc_sh.at[pl.ds(sid * RPT, RPT)])
                plsc.subcore_barrier()

                for h in range(2):
                    pltpu.sync_copy(
                        src_hbm.at[pl.ds(sid * NCH + h * half, half)], src_v)
                    pltpu.sync_copy(
                        dst_hbm.at[pl.ds(sid * NCH + h * half, half)], dst_v)

                    pltpu.async_copy(y_hbm.at[src_v.at[0]], gbuf0, sem0)

                    def body(i, carry):
                        j0 = 2 * i
                        pltpu.async_copy(
                            y_hbm.at[src_v.at[j0 + 1]], gbuf1, sem1)
                        pltpu.make_async_copy(
                            y_hbm.at[src_v.at[j0]], gbuf0, sem0).wait()
                        pltpu.sync_copy(
                            gbuf0, acc_sh.at[dst_v.at[j0]], add=True)

                        j2 = jnp.minimum(j0 + 2, half - 2)
                        pltpu.async_copy(y_hbm.at[src_v.at[j2]], gbuf0, sem0)

                        pltpu.make_async_copy(
                            y_hbm.at[src_v.at[j0 + 1]], gbuf1, sem1).wait()
                        pltpu.sync_copy(
                            gbuf1, acc_sh.at[dst_v.at[j0 + 1]], add=True)
                        return carry

                    lax.fori_loop(0, half // 2, body, 0)
                    pltpu.make_async_copy(
                        y_hbm.at[src_v.at[half - 2]], gbuf0, sem0).wait()

                plsc.subcore_barrier()

                @pl.when(sid < NT - 1)
                def _():
                    pltpu.sync_copy(acc_sh.at[pl.ds(sid * DR, DR)],
                                    out_hbm.at[pl.ds(sid * DR, DR)])

                @pl.when(sid == NT - 1)
                def _():
                    pltpu.sync_copy(
                        acc_sh.at[pl.ds((NT - 1) * DR, DR_LAST)],
                        out_hbm.at[pl.ds((NT - 1) * DR, DR_LAST)])

                plsc.subcore_barrier()

    return prop


def _scale(x, d0, d1):
    def body(x_ref, d0_ref, d1_ref, o0_ref, o1_ref):
        deg = d0_ref[...][:, 0:1] + d1_ref[...][:, 0:1] + 1.0
        dinv = lax.rsqrt(deg)
        o0_ref[...] = x_ref[...][:, :CHUNK] * dinv
        o1_ref[...] = x_ref[...][:, CHUNK:] * dinv

    return pl.pallas_call(
        body,
        grid=(N // MB,),
        in_specs=[
            pl.BlockSpec((MB, 2 * CHUNK), lambda i: (i, 0)),
            pl.BlockSpec((MB, 16), lambda i: (i, 0)),
            pl.BlockSpec((MB, 16), lambda i: (i, 0)),
        ],
        out_specs=[pl.BlockSpec((MB, CHUNK), lambda i: (i, 0))] * 2,
        out_shape=[jax.ShapeDtypeStruct((N, CHUNK), jnp.float32)] * 2,
    )(x, d0, d1)


def _layer1(s0, s1, y0, y1, d0, d1, W1, b1):
    def body(s0_ref, s1_ref, y0_ref, y1_ref, d0_ref, d1_ref, w_ref, b_ref,
             o0_ref, o1_ref, o2_ref, o3_ref):
        deg = d0_ref[...][:, 0:1] + d1_ref[...][:, 0:1] + 1.0
        dinv = lax.rsqrt(deg)
        p = jnp.concatenate(
            [s0_ref[...] + y0_ref[...], s1_ref[...] + y1_ref[...]],
            axis=1) * dinv
        h = jnp.dot(p, w_ref[...], preferred_element_type=jnp.float32)
        h = jnp.maximum(h + b_ref[...], 0.0) * dinv
        o0_ref[...] = h[:, 0 * CHUNK:1 * CHUNK]
        o1_ref[...] = h[:, 1 * CHUNK:2 * CHUNK]
        o2_ref[...] = h[:, 2 * CHUNK:3 * CHUNK]
        o3_ref[...] = h[:, 3 * CHUNK:4 * CHUNK]

    blk = pl.BlockSpec((MB, CHUNK), lambda i: (i, 0))
    return pl.pallas_call(
        body,
        grid=(N // MB,),
        in_specs=[
            blk, blk, blk, blk,
            pl.BlockSpec((MB, 16), lambda i: (i, 0)),
            pl.BlockSpec((MB, 16), lambda i: (i, 0)),
            pl.BlockSpec((256, 512), lambda i: (0, 0)),
            pl.BlockSpec((1, 512), lambda i: (0, 0)),
        ],
        out_specs=[blk] * 4,
        out_shape=[jax.ShapeDtypeStruct((N, CHUNK), jnp.float32)] * 4,
    )(s0, s1, y0, y1, d0, d1, W1, b1)


def _final(s, y, d0, d1, Wc, bc):
    def body(s0, s1, s2, s3, y0, y1, y2, y3, d0_ref, d1_ref, w_ref, b_ref,
             mu_ref, ls_ref):
        deg = d0_ref[...][:, 0:1] + d1_ref[...][:, 0:1] + 1.0
        dinv = lax.rsqrt(deg)
        srefs, yrefs = (s0, s1, s2, s3), (y0, y1, y2, y3)
        p = jnp.concatenate(
            [srefs[k][...] + yrefs[k][...] for k in range(4)], axis=1) * dinv
        o = jnp.dot(p, w_ref[...], preferred_element_type=jnp.float32)
        o = o + b_ref[...]
        mu_ref[...] = o[:, :256]
        ls_ref[...] = o[:, 256:]

    blk = pl.BlockSpec((MB, CHUNK), lambda i: (i, 0))
    return pl.pallas_call(
        body,
        grid=(N // MB,),
        in_specs=[blk] * 8 + [
            pl.BlockSpec((MB, 16), lambda i: (i, 0)),
            pl.BlockSpec((MB, 16), lambda i: (i, 0)),
            pl.BlockSpec((512, 512), lambda i: (0, 0)),
            pl.BlockSpec((1, 512), lambda i: (0, 0)),
        ],
        out_specs=[pl.BlockSpec((MB, 256), lambda i: (i, 0))] * 2,
        out_shape=[jax.ShapeDtypeStruct((N, 256), jnp.float32)] * 2,
    )(*s, *y, d0, d1, Wc, bc)


def kernel(x, edge_index, W1, b1, W_mu, b_mu, W_ls, b_ls):
    src = edge_index[0].astype(jnp.int32)
    dst = edge_index[1].astype(jnp.int32)
    pad = EPAD - E
    padi = jnp.arange(pad, dtype=jnp.int32) % 16
    src_p = jnp.concatenate([src, padi])
    dst_p = jnp.concatenate([dst, N + padi])
    src_flat = src_p.reshape(EPAD // CHUNK, CHUNK)
    dst_flat = dst_p.reshape(EPAD // CHUNK, CHUNK)
    ones128 = jnp.ones((CHUNK, CHUNK), jnp.float32)
    zeros128 = jnp.zeros((NACC, CHUNK), jnp.float32)

    degP = _make_deg_kernel()(dst_flat, ones128, zeros128)
    d0 = degP[:N, :16]
    d1 = degP[NACC:NACC + N, :16]

    y1 = _scale(x, d0, d1)

    S1 = _make_prop_kernel(2)(*y1, src_flat, dst_flat, zeros128)

    y2 = _layer1(*S1, *y1, d0, d1, W1, b1.reshape(1, 512))

    S2 = _make_prop_kernel(4)(*y2, src_flat, dst_flat, zeros128)

    Wc = jnp.concatenate([W_mu, W_ls], axis=1)
    bc = jnp.concatenate([b_mu, b_ls]).reshape(1, 512)
    mu, ls = _final(S2, y2, d0, d1, Wc, bc)
    return mu, ls
